# baseline (speedup 1.0000x reference)
import functools

import jax
import jax.numpy as jnp
from jax import lax
from jax.experimental import pallas as pl
from jax.experimental.pallas import tpu as pltpu
from jax.experimental.pallas import tpu_sc as plsc

_V = 100000
_VM = 99968
_VT = _V - _VM
_D = 300
_B = 4096
_S = 200
_NC = 2
_NS = 16
_NW = _NC * _NS
_CS = 8
_CB = 512
_NCH = (_S // _CS) * (_B // _CB)
_NPR = _NCH // 2


def _gather_body(idxT_hbm, tabT_hbm, tail3_hbm, dvals_hbm, outT_hbm,
                 row_m, tail_v, idx0, idx1, out0, out1, didx,
                 sem_r, sem_i0, sem_i1, sem_o0, sem_o1):
    cid = lax.axis_index("c")
    sid = lax.axis_index("s")
    w = sid * _NC + cid
    start = 9 * w + jnp.minimum(w, 12)
    n_rows = jnp.where(w < 12, 10, 9)

    lanes = lax.iota(jnp.int32, 16)
    zeros16 = lanes * 0

    def chunk_pos(c):
        s0 = (c // (_B // _CB)) * _CS
        b0 = (c % (_B // _CB)) * _CB
        return s0, b0

    def issue_idx(c, buf, sem):
        s0, b0 = chunk_pos(c)
        return pltpu.async_copy(
            idxT_hbm.at[pl.ds(s0, _CS), pl.ds(b0, _CB)], buf, sem)

    def compute(idx_v, out_v):
        def j_body(j, carry):
            for s in range(_CS):
                iv = idx_v[s, pl.ds(16 * j, 16)]
                g = plsc.load_gather(row_m.at[0], [jnp.minimum(iv, _VM - 1)])
                t = plsc.load_gather(tail_v.at[0], [jnp.maximum(iv - _VM, 0)])
                out_v[s, pl.ds(16 * j, 16)] = jnp.where(iv >= _VM, t, g)
            return carry
        lax.fori_loop(0, _CB // 16, j_body, 0)

    def issue_out(c, d, buf, sem):
        s0, b0 = chunk_pos(c)
        return pltpu.async_copy(
            buf, outT_hbm.at[d, pl.ds(s0, _CS), pl.ds(b0, _CB)], sem)

    def row_loop(r, carry):
        @pl.when(r < n_rows)
        def _():
            d = start + r
            pltpu.sync_copy(dvals_hbm.at[d, 0], didx)
            pltpu.async_copy(
                tabT_hbm.at[didx, pl.ds(0, _VM)],
                row_m, sem_r).wait()
            pltpu.sync_copy(tail3_hbm.at[d], tail_v)

            issue_idx(0, idx0, sem_i0)

            def pair_body(p, carry2):
                c0 = 2 * p
                c1 = c0 + 1
                gi1 = issue_idx(c1, idx1, sem_i1)
                s0, b0 = chunk_pos(c0)
                pltpu.make_async_copy(
                    idxT_hbm.at[pl.ds(s0, _CS), pl.ds(b0, _CB)],
                    idx0, sem_i0).wait()
                compute(idx0, out0)
                go0 = issue_out(c0, d, out0, sem_o0)

                @pl.when(p + 1 < _NPR)
                def _():
                    issue_idx(c0 + 2, idx0, sem_i0)

                gi1.wait()
                compute(idx1, out1)
                go1 = issue_out(c1, d, out1, sem_o1)
                go0.wait()
                go1.wait()
                return carry2

            lax.fori_loop(0, _NPR, pair_body, 0)
        return carry

    lax.fori_loop(0, 10, row_loop, 0)


@functools.partial(jax.jit, static_argnums=())
def kernel(data, table):
    idx_t = data.T.astype(jnp.int32)
    tab_t = table.T
    tail3 = tab_t[:, _VM:].reshape(_D, 1, _VT)
    dvals = jnp.arange(_D, dtype=jnp.int32).reshape(_D, 1, 1)
    mesh = plsc.VectorSubcoreMesh(
        core_axis_name="c", subcore_axis_name="s",
        num_cores=_NC, num_subcores=_NS)
    k = pl.kernel(
        _gather_body,
        out_type=jax.ShapeDtypeStruct((_D, _S, _B), jnp.float32),
        mesh=mesh,
        compiler_params=pltpu.CompilerParams(needs_layout_passes=False),
        scratch_types=[
            pltpu.VMEM((1, _VM), jnp.float32),
            pltpu.VMEM((1, _VT), jnp.float32),
            pltpu.VMEM((_CS, _CB), jnp.int32),
            pltpu.VMEM((_CS, _CB), jnp.int32),
            pltpu.VMEM((_CS, _CB), jnp.float32),
            pltpu.VMEM((_CS, _CB), jnp.float32),
            pltpu.VMEM((1,), jnp.int32),
            pltpu.SemaphoreType.DMA,
            pltpu.SemaphoreType.DMA,
            pltpu.SemaphoreType.DMA,
            pltpu.SemaphoreType.DMA,
            pltpu.SemaphoreType.DMA,
        ],
    )
    out_t = k(idx_t, tab_t, tail3, dvals)
    return out_t.transpose(2, 1, 0)
